# Initial kernel scaffold; baseline (speedup 1.0000x reference)
import jax
import jax.numpy as jnp
from jax.experimental import pallas as pl


def kernel(x, edge_index, W1l, b1l, W1r, W2l, b2l, W2r):
    raise NotImplementedError("write your pallas kernel here")


# speedup vs baseline: 3.1057x; 3.1057x over previous
import functools

import jax
import jax.numpy as jnp
from jax import lax
from jax.experimental import pallas as pl
from jax.experimental.pallas import tpu as pltpu
from jax.experimental.pallas import tpu_sc as plsc

N_NODES = 10000
D = 128
E = 320000

NC = 2
NS = 16
NW = NC * NS

CHUNK = 128
CH_PER_W = 80
EPW = CHUNK * CH_PER_W
E_PAD = EPW * NW
N_PAD = 10112
ROWS_PER_TILE = N_PAD // NS


def _sc_agg_body(table, src_i, dst_i, zrows, agg_out,
                 src_v, dst_v, rows_v, agg_sh, sem):
    c = lax.axis_index("c")
    s = lax.axis_index("s")
    wid = c * NS + s
    r0 = s * ROWS_PER_TILE

    pltpu.sync_copy(zrows.at[pl.ds(r0, ROWS_PER_TILE)],
                    agg_sh.at[pl.ds(r0, ROWS_PER_TILE)])
    pltpu.sync_copy(src_i.at[pl.ds(wid * CH_PER_W, CH_PER_W)], src_v)
    pltpu.sync_copy(dst_i.at[pl.ds(wid * CH_PER_W, CH_PER_W)], dst_v)

    plsc.subcore_barrier()

    def step(j, carry):
        pltpu.async_copy(table.at[src_v.at[j]], rows_v, sem).wait()
        pltpu.sync_copy(rows_v, agg_sh.at[dst_v.at[j]], add=True)
        return carry

    lax.fori_loop(0, CH_PER_W, step, 0)

    plsc.subcore_barrier()

    pltpu.sync_copy(agg_sh.at[pl.ds(r0, ROWS_PER_TILE)],
                    agg_out.at[c, pl.ds(r0, ROWS_PER_TILE)])


def _sc_cnt_body(dst_i, zrows, ones_h, cnt_out, dst_v, ones_v, cnt_sh):
    c = lax.axis_index("c")
    s = lax.axis_index("s")
    wid = c * NS + s
    r0 = s * ROWS_PER_TILE

    pltpu.sync_copy(zrows.at[pl.ds(r0, ROWS_PER_TILE)],
                    cnt_sh.at[pl.ds(r0, ROWS_PER_TILE)])
    pltpu.sync_copy(ones_h, ones_v)
    pltpu.sync_copy(dst_i.at[pl.ds(wid * CH_PER_W, CH_PER_W)], dst_v)

    plsc.subcore_barrier()

    def step(j, carry):
        pltpu.sync_copy(ones_v, cnt_sh.at[dst_v.at[j]], add=True)
        return carry

    lax.fori_loop(0, CH_PER_W, step, 0)

    plsc.subcore_barrier()

    pltpu.sync_copy(cnt_sh.at[pl.ds(r0, ROWS_PER_TILE)],
                    cnt_out.at[c, pl.ds(r0, ROWS_PER_TILE)])


@functools.lru_cache(maxsize=None)
def _make_sc_kernels():
    mesh = plsc.VectorSubcoreMesh(core_axis_name="c", subcore_axis_name="s",
                                  num_cores=NC, num_subcores=NS)
    agg = pl.kernel(
        _sc_agg_body,
        out_type=[jax.ShapeDtypeStruct((NC, N_PAD, D), jnp.float32)],
        mesh=mesh,
        scratch_types=[
            pltpu.VMEM((CH_PER_W, CHUNK), jnp.int32),
            pltpu.VMEM((CH_PER_W, CHUNK), jnp.int32),
            pltpu.VMEM((CHUNK, D), jnp.float32),
            pltpu.VMEM_SHARED((N_PAD, D), jnp.float32),
            pltpu.SemaphoreType.DMA,
        ],
    )
    cnt = pl.kernel(
        _sc_cnt_body,
        out_type=[jax.ShapeDtypeStruct((NC, N_PAD, D), jnp.float32)],
        mesh=mesh,
        scratch_types=[
            pltpu.VMEM((CH_PER_W, CHUNK), jnp.int32),
            pltpu.VMEM((CHUNK, D), jnp.float32),
            pltpu.VMEM_SHARED((N_PAD, D), jnp.float32),
        ],
    )
    return agg, cnt


def _tc_body(relu, agg_ref, cnt_ref, x_ref, wl_ref, bl_ref, wr_ref, out_ref):
    agg = agg_ref[0, :N_NODES, :] + agg_ref[1, :N_NODES, :]
    cnt = cnt_ref[0, :N_NODES, 0:1] + cnt_ref[1, :N_NODES, 0:1]
    mean = agg / jnp.maximum(cnt, 1.0)
    out = lax.dot_general(mean, wl_ref[...], (((1,), (1,)), ((), ())),
                          preferred_element_type=jnp.float32)
    out = out + bl_ref[...][None, :]
    out = out + lax.dot_general(x_ref[...], wr_ref[...], (((1,), (1,)), ((), ())),
                                preferred_element_type=jnp.float32)
    if relu:
        out = jnp.maximum(out, 0.0)
    out_ref[...] = out


def _tc_layer(relu):
    return pl.pallas_call(
        functools.partial(_tc_body, relu),
        out_shape=jax.ShapeDtypeStruct((N_NODES, D), jnp.float32),
    )


_tc1 = _tc_layer(True)
_tc2 = _tc_layer(False)


def kernel(x, edge_index, W1l, b1l, W1r, W2l, b2l, W2r):
    src = edge_index[0].astype(jnp.int32)
    dst = edge_index[1].astype(jnp.int32)
    src = jnp.pad(src, (0, E_PAD - E)).reshape(NW * CH_PER_W, CHUNK)
    dst = jnp.pad(dst, (0, E_PAD - E),
                  constant_values=N_NODES).reshape(NW * CH_PER_W, CHUNK)
    zrows = jnp.zeros((N_PAD, D), jnp.float32)
    ones = jnp.ones((CHUNK, D), jnp.float32)

    sc_agg, sc_cnt = _make_sc_kernels()
    (cnt,) = sc_cnt(dst, zrows, ones)
    (agg1,) = sc_agg(x, src, dst, zrows)
    h = _tc1(agg1, cnt, x, W1l, b1l, W1r)
    (agg2,) = sc_agg(h, src, dst, zrows)
    out = _tc2(agg2, cnt, h, W2l, b2l, W2r)
    return out


# speedup vs baseline: 3.1805x; 1.0241x over previous
import functools

import jax
import jax.numpy as jnp
from jax import lax
from jax.experimental import pallas as pl
from jax.experimental.pallas import tpu as pltpu
from jax.experimental.pallas import tpu_sc as plsc

N_NODES = 10000
D = 128
E = 320000

NC = 2
NS = 16
NW = NC * NS

CHUNK = 64
CH_PER_W = 160
EPW = CHUNK * CH_PER_W
E_PAD = EPW * NW
N_PAD = 10112
ROWS_PER_TILE = N_PAD // NS


def _sc_agg_body(table, src_i, dst_i, zrows, agg_out,
                 src_v, dst_a, dst_b, rows_a, rows_b, agg_sh,
                 sem_a, sem_b, dsem_a, dsem_b):
    c = lax.axis_index("c")
    s = lax.axis_index("s")
    wid = c * NS + s
    r0 = s * ROWS_PER_TILE
    row0 = wid * CH_PER_W

    pltpu.sync_copy(zrows.at[pl.ds(r0, ROWS_PER_TILE)],
                    agg_sh.at[pl.ds(r0, ROWS_PER_TILE)])
    pltpu.sync_copy(src_i.at[pl.ds(row0, CH_PER_W)], src_v)

    plsc.subcore_barrier()

    bufs = (rows_a, rows_b)
    sems = (sem_a, sem_b)
    dbufs = (dst_a, dst_b)
    dsems = (dsem_a, dsem_b)

    pltpu.async_copy(table.at[src_v.at[0]], rows_a, sem_a)
    pltpu.async_copy(dst_i.at[pl.ds(row0, 1)], dst_a, dsem_a)

    def outer(i, carry):
        for b in range(2):
            j = 2 * i + b
            jn = jnp.minimum(j + 1, CH_PER_W - 1)
            pltpu.async_copy(table.at[src_v.at[jn]], bufs[1 - b], sems[1 - b])
            pltpu.async_copy(dst_i.at[pl.ds(row0 + jn, 1)], dbufs[1 - b],
                             dsems[1 - b])
            pltpu.make_async_copy(table.at[src_v.at[j]], bufs[b], sems[b]).wait()
            pltpu.make_async_copy(dst_i.at[pl.ds(row0, 1)], dbufs[b],
                                  dsems[b]).wait()
            pltpu.sync_copy(bufs[b], agg_sh.at[dbufs[b].at[0]], add=True)
        return carry

    lax.fori_loop(0, CH_PER_W // 2, outer, 0)
    pltpu.make_async_copy(table.at[src_v.at[0]], rows_a, sem_a).wait()
    pltpu.make_async_copy(dst_i.at[pl.ds(row0, 1)], dst_a, dsem_a).wait()

    plsc.subcore_barrier()

    pltpu.sync_copy(agg_sh.at[pl.ds(r0, ROWS_PER_TILE)],
                    agg_out.at[c, pl.ds(r0, ROWS_PER_TILE)])


def _sc_cnt_body(dst_i, zrows, ones_h, cnt_out, dst_v, ones_v, cnt_sh):
    c = lax.axis_index("c")
    s = lax.axis_index("s")
    wid = c * NS + s
    r0 = s * ROWS_PER_TILE

    pltpu.sync_copy(zrows.at[pl.ds(r0, ROWS_PER_TILE)],
                    cnt_sh.at[pl.ds(r0, ROWS_PER_TILE)])
    pltpu.sync_copy(ones_h, ones_v)
    pltpu.sync_copy(dst_i.at[pl.ds(wid * CH_PER_W, CH_PER_W)], dst_v)

    plsc.subcore_barrier()

    def step(j, carry):
        pltpu.sync_copy(ones_v, cnt_sh.at[dst_v.at[j]], add=True)
        return carry

    lax.fori_loop(0, CH_PER_W, step, 0)

    plsc.subcore_barrier()

    pltpu.sync_copy(cnt_sh.at[pl.ds(r0, ROWS_PER_TILE)],
                    cnt_out.at[c, pl.ds(r0, ROWS_PER_TILE)])


@functools.lru_cache(maxsize=None)
def _make_sc_kernels():
    mesh = plsc.VectorSubcoreMesh(core_axis_name="c", subcore_axis_name="s",
                                  num_cores=NC, num_subcores=NS)
    agg = pl.kernel(
        _sc_agg_body,
        out_type=[jax.ShapeDtypeStruct((NC, N_PAD, D), jnp.float32)],
        mesh=mesh,
        scratch_types=[
            pltpu.VMEM((CH_PER_W, CHUNK), jnp.int32),
            pltpu.VMEM((1, CHUNK), jnp.int32),
            pltpu.VMEM((1, CHUNK), jnp.int32),
            pltpu.VMEM((CHUNK, D), jnp.float32),
            pltpu.VMEM((CHUNK, D), jnp.float32),
            pltpu.VMEM_SHARED((N_PAD, D), jnp.float32),
            pltpu.SemaphoreType.DMA,
            pltpu.SemaphoreType.DMA,
            pltpu.SemaphoreType.DMA,
            pltpu.SemaphoreType.DMA,
        ],
    )
    cnt = pl.kernel(
        _sc_cnt_body,
        out_type=[jax.ShapeDtypeStruct((NC, N_PAD, D), jnp.float32)],
        mesh=mesh,
        scratch_types=[
            pltpu.VMEM((CH_PER_W, CHUNK), jnp.int32),
            pltpu.VMEM((CHUNK, D), jnp.float32),
            pltpu.VMEM_SHARED((N_PAD, D), jnp.float32),
        ],
    )
    return agg, cnt


def _tc_body(relu, agg_ref, cnt_ref, x_ref, wl_ref, bl_ref, wr_ref, out_ref):
    agg = agg_ref[0, :N_NODES, :] + agg_ref[1, :N_NODES, :]
    cnt = cnt_ref[0, :N_NODES, 0:1] + cnt_ref[1, :N_NODES, 0:1]
    mean = agg / jnp.maximum(cnt, 1.0)
    out = lax.dot_general(mean, wl_ref[...], (((1,), (1,)), ((), ())),
                          preferred_element_type=jnp.float32)
    out = out + bl_ref[...][None, :]
    out = out + lax.dot_general(x_ref[...], wr_ref[...], (((1,), (1,)), ((), ())),
                                preferred_element_type=jnp.float32)
    if relu:
        out = jnp.maximum(out, 0.0)
    out_ref[...] = out


def _tc_layer(relu):
    return pl.pallas_call(
        functools.partial(_tc_body, relu),
        out_shape=jax.ShapeDtypeStruct((N_NODES, D), jnp.float32),
    )


_tc1 = _tc_layer(True)
_tc2 = _tc_layer(False)


def kernel(x, edge_index, W1l, b1l, W1r, W2l, b2l, W2r):
    src = edge_index[0].astype(jnp.int32)
    dst = edge_index[1].astype(jnp.int32)
    src = jnp.pad(src, (0, E_PAD - E)).reshape(NW * CH_PER_W, CHUNK)
    dst = jnp.pad(dst, (0, E_PAD - E),
                  constant_values=N_NODES).reshape(NW * CH_PER_W, CHUNK)
    zrows = jnp.zeros((N_PAD, D), jnp.float32)
    ones = jnp.ones((CHUNK, D), jnp.float32)

    sc_agg, sc_cnt = _make_sc_kernels()
    (cnt,) = sc_cnt(dst, zrows, ones)
    (agg1,) = sc_agg(x, src, dst, zrows)
    h = _tc1(agg1, cnt, x, W1l, b1l, W1r)
    (agg2,) = sc_agg(h, src, dst, zrows)
    out = _tc2(agg2, cnt, h, W2l, b2l, W2r)
    return out
